# Initial kernel scaffold; baseline (speedup 1.0000x reference)
import functools
import math

import jax
import jax.numpy as jnp
import numpy as np
from jax import lax
from jax.experimental import pallas as pl
from jax.experimental.pallas import tpu as pltpu
from jax.experimental.pallas import tpu_sc as plsc

VOCAB = 1000000
D = 64
L_SEQ = 200
B = 4096
N = B * L_SEQ
NC, NS, LANES = 2, 16, 16
NW = NC * NS
ROWS_PER_W = N // NW
G = 128
CHUNK_G = 4
CHUNK = G * CHUNK_G
N_CHUNKS = ROWS_PER_W // CHUNK


def _pos_embed_np(d_model, max_len):
    pos = np.arange(0, max_len).reshape(-1, 1).astype(np.float32)
    div_term = np.exp(
        np.arange(0, d_model, 2).astype(np.float32) * -(math.log(10000.0) / d_model))
    pe = np.zeros((max_len, d_model), dtype=np.float32)
    pe[:, 0::2] = np.sin(pos * div_term)
    pe[:, 1::2] = np.cos(pos * div_term)
    return pe


_mesh = plsc.VectorSubcoreMesh(core_axis_name="c", subcore_axis_name="s",
                               num_cores=NC, num_subcores=NS)


@functools.partial(
    pl.kernel,
    out_type=jax.ShapeDtypeStruct((N, D), jnp.float32),
    mesh=_mesh,
    scratch_types=[
        pltpu.VMEM((CHUNK_G, G), jnp.int32),
        pltpu.VMEM((CHUNK_G, G), jnp.int32),
        pltpu.VMEM((CHUNK_G, G), jnp.int32),
        pltpu.VMEM((CHUNK, D), jnp.float32),
        pltpu.VMEM((CHUNK, D), jnp.float32),
        pltpu.SemaphoreType.DMA,
        pltpu.SemaphoreType.DMA,
    ],
)
def _embed_kernel(tok_hbm, add_hbm, seq_hbm, seg_hbm, out_hbm,
                  seqb, segb, aidxb, tokb, addb, sem_t, sem_a):
    wid = lax.axis_index("s") * NC + lax.axis_index("c")
    gw = wid * (ROWS_PER_W // G)
    iota16 = lax.iota(jnp.int32, 16)

    def chunk_body(c, _):
        gbase = gw + c * CHUNK_G
        rowbase = gbase * G
        pltpu.sync_copy(seq_hbm.at[pl.ds(gbase, CHUNK_G)], seqb)
        pltpu.sync_copy(seg_hbm.at[pl.ds(gbase, CHUNK_G)], segb)

        for g in range(CHUNK_G):
            for k in range(G // LANES):
                off = k * LANES
                rows = rowbase + g * G + off + iota16
                lmod = rows % L_SEQ
                seqv = seqb[g, pl.ds(off, LANES)]
                segv = segb[g, pl.ds(off, LANES)]
                aidx = lmod + segv * L_SEQ + jnp.where(
                    seqv == 0, jnp.int32(2 * L_SEQ), jnp.int32(0))
                aidxb[g, pl.ds(off, LANES)] = aidx

        descs = []
        for g in range(CHUNK_G):
            descs.append(pltpu.async_copy(
                tok_hbm.at[seqb.at[g]], tokb.at[pl.ds(g * G, G)], sem_t))
            descs.append(pltpu.async_copy(
                add_hbm.at[aidxb.at[g]], addb.at[pl.ds(g * G, G)], sem_a))
        for d in descs:
            d.wait()

        def add_row(r, _):
            for k in range(D // LANES):
                sl = pl.ds(k * LANES, LANES)
                tokb[r, sl] = tokb[r, sl] + addb[r, sl]
            return 0
        lax.fori_loop(0, CHUNK, add_row, 0)

        pltpu.sync_copy(tokb, out_hbm.at[pl.ds(rowbase, CHUNK)])
        return 0

    lax.fori_loop(0, N_CHUNKS, chunk_body, 0)


def kernel(sequence, segment_label, token_table, segment_table):
    seq = sequence.astype(jnp.int32).reshape(N // G, G)
    seg = segment_label.astype(jnp.int32).reshape(N // G, G)

    pe = jnp.asarray(_pos_embed_np(D, L_SEQ))
    seg1 = segment_table[1][None, :]
    tok0 = token_table[0][None, :]
    add_tbl = jnp.concatenate(
        [pe, pe + seg1, pe - tok0, pe + seg1 - tok0], axis=0)

    out = _embed_kernel(token_table, add_tbl, seq, seg)
    return out.reshape(B, L_SEQ, D)


# speedup vs baseline: 2.2112x; 2.2112x over previous
import functools
import math

import jax
import jax.numpy as jnp
import numpy as np
from jax import lax
from jax.experimental import pallas as pl
from jax.experimental.pallas import tpu as pltpu
from jax.experimental.pallas import tpu_sc as plsc

VOCAB = 1000000
D = 64
L_SEQ = 200
B = 4096
N = B * L_SEQ
NC, NS, LANES = 2, 16, 16
NW = NC * NS
ROWS_PER_W = N // NW
G = 128
CHUNK_G = 4
CHUNK = G * CHUNK_G
N_CHUNKS = ROWS_PER_W // CHUNK


def _pos_embed_np(d_model, max_len):
    pos = np.arange(0, max_len).reshape(-1, 1).astype(np.float32)
    div_term = np.exp(
        np.arange(0, d_model, 2).astype(np.float32) * -(math.log(10000.0) / d_model))
    pe = np.zeros((max_len, d_model), dtype=np.float32)
    pe[:, 0::2] = np.sin(pos * div_term)
    pe[:, 1::2] = np.cos(pos * div_term)
    return pe


_mesh = plsc.VectorSubcoreMesh(core_axis_name="c", subcore_axis_name="s",
                               num_cores=NC, num_subcores=NS)


@functools.partial(
    pl.kernel,
    out_type=jax.ShapeDtypeStruct((N, D), jnp.float32),
    mesh=_mesh,
    compiler_params=pltpu.CompilerParams(use_tc_tiling_on_sc=False),
    scratch_types=[
        pltpu.VMEM((CHUNK_G, G), jnp.int32),
        pltpu.VMEM((CHUNK_G, G), jnp.int32),
        pltpu.VMEM((CHUNK_G, G), jnp.int32),
        pltpu.VMEM((CHUNK, D), jnp.float32),
        pltpu.VMEM((CHUNK, D), jnp.float32),
        pltpu.SemaphoreType.DMA,
        pltpu.SemaphoreType.DMA,
    ],
)
def _embed_kernel(tok_hbm, add_hbm, seq_hbm, seg_hbm, out_hbm,
                  seqb, segb, aidxb, tokb, addb, sem_t, sem_a):
    wid = lax.axis_index("s") * NC + lax.axis_index("c")
    gw = wid * (ROWS_PER_W // G)
    iota16 = lax.iota(jnp.int32, 16)

    def chunk_body(c, _):
        gbase = gw + c * CHUNK_G
        rowbase = gbase * G
        pltpu.sync_copy(seq_hbm.at[pl.ds(gbase, CHUNK_G)], seqb)
        pltpu.sync_copy(seg_hbm.at[pl.ds(gbase, CHUNK_G)], segb)

        for g in range(CHUNK_G):
            for k in range(G // LANES):
                off = k * LANES
                rows = rowbase + g * G + off + iota16
                lmod = rows % L_SEQ
                seqv = seqb[g, pl.ds(off, LANES)]
                segv = segb[g, pl.ds(off, LANES)]
                aidx = lmod + segv * L_SEQ + jnp.where(
                    seqv == 0, jnp.int32(2 * L_SEQ), jnp.int32(0))
                aidxb[g, pl.ds(off, LANES)] = aidx

        descs = []
        for g in range(CHUNK_G):
            descs.append(pltpu.async_copy(
                tok_hbm.at[seqb.at[g]], tokb.at[pl.ds(g * G, G)], sem_t))
            descs.append(pltpu.async_copy(
                add_hbm.at[aidxb.at[g]], addb.at[pl.ds(g * G, G)], sem_a))
        for d in descs:
            d.wait()

        def add_row(r, _):
            for k in range(D // LANES):
                sl = pl.ds(k * LANES, LANES)
                tokb[r, sl] = tokb[r, sl] + addb[r, sl]
            return 0
        lax.fori_loop(0, CHUNK, add_row, 0)

        pltpu.sync_copy(tokb, out_hbm.at[pl.ds(rowbase, CHUNK)])
        return 0

    lax.fori_loop(0, N_CHUNKS, chunk_body, 0)


def kernel(sequence, segment_label, token_table, segment_table):
    seq = sequence.astype(jnp.int32).reshape(N // G, G)
    seg = segment_label.astype(jnp.int32).reshape(N // G, G)

    pe = jnp.asarray(_pos_embed_np(D, L_SEQ))
    seg1 = segment_table[1][None, :]
    tok0 = token_table[0][None, :]
    add_tbl = jnp.concatenate(
        [pe, pe + seg1, pe - tok0, pe + seg1 - tok0], axis=0)

    out = _embed_kernel(token_table, add_tbl, seq, seg)
    return out.reshape(B, L_SEQ, D)


# speedup vs baseline: 2.2236x; 1.0056x over previous
import functools
import math

import jax
import jax.numpy as jnp
import numpy as np
from jax import lax
from jax.experimental import pallas as pl
from jax.experimental.pallas import tpu as pltpu
from jax.experimental.pallas import tpu_sc as plsc

VOCAB = 1000000
D = 64
L_SEQ = 200
B = 4096
N = B * L_SEQ
NC, NS, LANES = 2, 16, 16
NW = NC * NS
ROWS_PER_W = N // NW
G = 128
CHUNK_G = 2
CHUNK = G * CHUNK_G
N_CHUNKS = ROWS_PER_W // CHUNK


def _pos_embed_np(d_model, max_len):
    pos = np.arange(0, max_len).reshape(-1, 1).astype(np.float32)
    div_term = np.exp(
        np.arange(0, d_model, 2).astype(np.float32) * -(math.log(10000.0) / d_model))
    pe = np.zeros((max_len, d_model), dtype=np.float32)
    pe[:, 0::2] = np.sin(pos * div_term)
    pe[:, 1::2] = np.cos(pos * div_term)
    return pe


_mesh = plsc.VectorSubcoreMesh(core_axis_name="c", subcore_axis_name="s",
                               num_cores=NC, num_subcores=NS)


@functools.partial(
    pl.kernel,
    out_type=jax.ShapeDtypeStruct((N, D), jnp.float32),
    mesh=_mesh,
    compiler_params=pltpu.CompilerParams(use_tc_tiling_on_sc=False),
    scratch_types=[
        pltpu.VMEM((2, CHUNK_G, G), jnp.int32),
        pltpu.VMEM((2, CHUNK_G, G), jnp.int32),
        pltpu.VMEM((2, CHUNK_G, G), jnp.int32),
        pltpu.VMEM((2, CHUNK, D), jnp.float32),
        pltpu.VMEM((2, CHUNK, D), jnp.float32),
        pltpu.SemaphoreType.DMA,
        pltpu.SemaphoreType.DMA,
        pltpu.SemaphoreType.DMA,
        pltpu.SemaphoreType.DMA,
        pltpu.SemaphoreType.DMA,
        pltpu.SemaphoreType.DMA,
    ],
)
def _embed_kernel(tok_hbm, add_hbm, seq_hbm, seg_hbm, out_hbm,
                  seqb, segb, aidxb, tokb, addb,
                  sem_t0, sem_t1, sem_a0, sem_a1, sem_o0, sem_o1):
    sem_t = (sem_t0, sem_t1)
    sem_a = (sem_a0, sem_a1)
    sem_o = (sem_o0, sem_o1)
    wid = lax.axis_index("s") * NC + lax.axis_index("c")
    gw = wid * (ROWS_PER_W // G)
    iota16 = lax.iota(jnp.int32, 16)

    def prefetch_fire(c, p, drain_out):
        gbase = gw + c * CHUNK_G
        rowbase = gbase * G
        pltpu.sync_copy(seq_hbm.at[pl.ds(gbase, CHUNK_G)], seqb.at[p])
        pltpu.sync_copy(seg_hbm.at[pl.ds(gbase, CHUNK_G)], segb.at[p])

        for g in range(CHUNK_G):
            for k in range(G // LANES):
                off = k * LANES
                rows = rowbase + g * G + off + iota16
                lmod = rows % L_SEQ
                seqv = seqb[p, g, pl.ds(off, LANES)]
                segv = segb[p, g, pl.ds(off, LANES)]
                aidx = lmod + segv * L_SEQ + jnp.where(
                    seqv == 0, jnp.int32(2 * L_SEQ), jnp.int32(0))
                aidxb[p, g, pl.ds(off, LANES)] = aidx

        if drain_out:
            pltpu.make_async_copy(
                tokb.at[p], out_hbm.at[pl.ds(0, CHUNK)], sem_o[p]).wait()

        for g in range(CHUNK_G):
            pltpu.async_copy(
                tok_hbm.at[seqb.at[p, g]],
                tokb.at[p, pl.ds(g * G, G)], sem_t[p])
            pltpu.async_copy(
                add_hbm.at[aidxb.at[p, g]],
                addb.at[p, pl.ds(g * G, G)], sem_a[p])

    def wait_add_out(c, p):
        for g in range(CHUNK_G):
            pltpu.make_async_copy(
                tok_hbm.at[seqb.at[p, g]],
                tokb.at[p, pl.ds(g * G, G)], sem_t[p]).wait()
            pltpu.make_async_copy(
                add_hbm.at[aidxb.at[p, g]],
                addb.at[p, pl.ds(g * G, G)], sem_a[p]).wait()

        def add_row(r, _):
            for k in range(D // LANES):
                sl = pl.ds(k * LANES, LANES)
                tokb[p, r, sl] = tokb[p, r, sl] + addb[p, r, sl]
            return 0
        lax.fori_loop(0, CHUNK, add_row, 0)

        rowbase = (gw + c * CHUNK_G) * G
        pltpu.async_copy(tokb.at[p], out_hbm.at[pl.ds(rowbase, CHUNK)],
                         sem_o[p])

    prefetch_fire(jnp.int32(0), 0, False)
    prefetch_fire(jnp.int32(1), 1, False)

    def pair_body(cc, _):
        for b in (0, 1):
            c = cc * 2 + b
            wait_add_out(c, b)

            @pl.when(c + 2 < N_CHUNKS)
            def _():
                prefetch_fire(c + 2, b, True)
        return 0

    lax.fori_loop(0, N_CHUNKS // 2, pair_body, 0)

    pltpu.make_async_copy(tokb.at[0], out_hbm.at[pl.ds(0, CHUNK)],
                          sem_o0).wait()
    pltpu.make_async_copy(tokb.at[1], out_hbm.at[pl.ds(0, CHUNK)],
                          sem_o1).wait()


def kernel(sequence, segment_label, token_table, segment_table):
    seq = sequence.astype(jnp.int32).reshape(N // G, G)
    seg = segment_label.astype(jnp.int32).reshape(N // G, G)

    pe = jnp.asarray(_pos_embed_np(D, L_SEQ))
    seg1 = segment_table[1][None, :]
    tok0 = token_table[0][None, :]
    add_tbl = jnp.concatenate(
        [pe, pe + seg1, pe - tok0, pe + seg1 - tok0], axis=0)

    out = _embed_kernel(token_table, add_tbl, seq, seg)
    return out.reshape(B, L_SEQ, D)


# speedup vs baseline: 2.5269x; 1.1364x over previous
import functools
import math

import jax
import jax.numpy as jnp
import numpy as np
from jax import lax
from jax.experimental import pallas as pl
from jax.experimental.pallas import tpu as pltpu
from jax.experimental.pallas import tpu_sc as plsc

VOCAB = 1000000
D = 64
DP = 128
L_SEQ = 200
B = 4096
N = B * L_SEQ
NC, NS, LANES = 2, 16, 16
NW = NC * NS
ROWS_PER_W = N // NW
G = 128
N_CHUNKS = ROWS_PER_W // G
GS = G // 8


def _pos_embed_np(d_model, max_len):
    pos = np.arange(0, max_len).reshape(-1, 1).astype(np.float32)
    div_term = np.exp(
        np.arange(0, d_model, 2).astype(np.float32) * -(math.log(10000.0) / d_model))
    pe = np.zeros((max_len, d_model), dtype=np.float32)
    pe[:, 0::2] = np.sin(pos * div_term)
    pe[:, 1::2] = np.cos(pos * div_term)
    return pe


_mesh = plsc.VectorSubcoreMesh(core_axis_name="c", subcore_axis_name="s",
                               num_cores=NC, num_subcores=NS)


@functools.partial(
    pl.kernel,
    out_type=jax.ShapeDtypeStruct((N // 8, 8, D), jnp.float32),
    mesh=_mesh,
    scratch_types=[
        pltpu.VMEM((2, G), jnp.int32),
        pltpu.VMEM((2, G), jnp.int32),
        pltpu.VMEM((2, G), jnp.int32),
        pltpu.VMEM((2, G, DP), jnp.float32),
        pltpu.VMEM((2, G, DP), jnp.float32),
        pltpu.VMEM((2, GS, 8, D), jnp.float32),
        pltpu.SemaphoreType.DMA,
        pltpu.SemaphoreType.DMA,
        pltpu.SemaphoreType.DMA,
        pltpu.SemaphoreType.DMA,
        pltpu.SemaphoreType.DMA,
        pltpu.SemaphoreType.DMA,
    ],
)
def _embed_kernel(tok_hbm, add_hbm, seq_hbm, seg_hbm, out_hbm,
                  seqb, segb, aidxb, tokb, addb, outb,
                  sem_t0, sem_t1, sem_a0, sem_a1, sem_o0, sem_o1):
    sem_t = (sem_t0, sem_t1)
    sem_a = (sem_a0, sem_a1)
    sem_o = (sem_o0, sem_o1)
    wid = lax.axis_index("s") * NC + lax.axis_index("c")
    gw = wid * N_CHUNKS
    iota16 = lax.iota(jnp.int32, 16)

    def prefetch_fire(c, p):
        gbase = gw + c
        rowbase = gbase * G
        pltpu.sync_copy(seq_hbm.at[gbase], seqb.at[p])
        pltpu.sync_copy(seg_hbm.at[gbase], segb.at[p])

        for k in range(G // LANES):
            off = k * LANES
            rows = rowbase + off + iota16
            lmod = rows % L_SEQ
            seqv = seqb[p, pl.ds(off, LANES)]
            segv = segb[p, pl.ds(off, LANES)]
            aidx = lmod + segv * L_SEQ + jnp.where(
                seqv == 0, jnp.int32(2 * L_SEQ), jnp.int32(0))
            aidxb[p, pl.ds(off, LANES)] = aidx

        pltpu.async_copy(tok_hbm.at[seqb.at[p]], tokb.at[p], sem_t[p])
        pltpu.async_copy(add_hbm.at[aidxb.at[p]], addb.at[p], sem_a[p])

    def wait_add_out(c, p, drain_out):
        pltpu.make_async_copy(
            tok_hbm.at[seqb.at[p]], tokb.at[p], sem_t[p]).wait()
        pltpu.make_async_copy(
            add_hbm.at[aidxb.at[p]], addb.at[p], sem_a[p]).wait()

        if drain_out:
            pltpu.make_async_copy(
                outb.at[p], out_hbm.at[pl.ds(0, GS)], sem_o[p]).wait()

        def add_slab(s, _):
            for j in range(8):
                for k in range(D // LANES):
                    sl = pl.ds(k * LANES, LANES)
                    outb[p, s, j, sl] = (
                        tokb[p, s * 8 + j, sl] + addb[p, s * 8 + j, sl])
            return 0
        lax.fori_loop(0, GS, add_slab, 0)

        sbase = (gw + c) * GS
        pltpu.async_copy(outb.at[p], out_hbm.at[pl.ds(sbase, GS)], sem_o[p])

    prefetch_fire(jnp.int32(0), 0)
    prefetch_fire(jnp.int32(1), 1)

    def pair_body(cc, _):
        for b in (0, 1):
            c = cc * 2 + b
            wait_add_out(c, b, True)

            @pl.when(c + 2 < N_CHUNKS)
            def _():
                prefetch_fire(c + 2, b)
        return 0

    for b in (0, 1):
        wait_add_out(jnp.int32(b), b, False)
        prefetch_fire(jnp.int32(2 + b), b)

    lax.fori_loop(1, N_CHUNKS // 2, pair_body, 0)

    pltpu.make_async_copy(outb.at[0], out_hbm.at[pl.ds(0, GS)], sem_o0).wait()
    pltpu.make_async_copy(outb.at[1], out_hbm.at[pl.ds(0, GS)], sem_o1).wait()


def kernel(sequence, segment_label, token_table, segment_table):
    seq = sequence.astype(jnp.int32).reshape(N // G, G)
    seg = segment_label.astype(jnp.int32).reshape(N // G, G)

    tokp = jnp.pad(token_table, ((0, 0), (0, DP - D)))

    pe = jnp.asarray(_pos_embed_np(D, L_SEQ))
    seg1 = segment_table[1][None, :]
    tok0 = token_table[0][None, :]
    add_tbl = jnp.concatenate(
        [pe, pe + seg1, pe - tok0, pe + seg1 - tok0], axis=0)
    add_tbl = jnp.pad(add_tbl, ((0, 0), (0, DP - D)))

    out = _embed_kernel(tokp, add_tbl, seq, seg)
    return out.reshape(B, L_SEQ, D)
